# baseline (speedup 1.0000x reference)
import jax
import jax.numpy as jnp
from jax import lax
from jax.experimental import pallas as pl
from jax.experimental.pallas import tpu as pltpu

BASE_SCALE = 150.0
BASE_SHIFT = 10.0
H, W = 2160, 3840
BH = 360


def _body(params_ref, raw_ref, out_ref):
    p = params_ref
    q0, q1, q2, q3 = p[0], p[1], p[2], p[3]
    t0, t1, t2 = p[4], p[5], p[6]
    scale, shift = p[7], p[8]
    fx, fy, ox, oy = p[9], p[10], p[11], p[12]

    inv = lax.rsqrt(q0 * q0 + q1 * q1 + q2 * q2 + q3 * q3 + 1e-12)
    w = q0 * inv
    x = q1 * inv
    y = q2 * inv
    z = q3 * inv
    r00 = 1 - 2 * (y * y + z * z)
    r01 = 2 * (x * y - w * z)
    r02 = 2 * (x * z + w * y)
    r10 = 2 * (x * y + w * z)
    r11 = 1 - 2 * (x * x + z * z)
    r12 = 2 * (y * z - w * x)
    r20 = 2 * (x * z - w * y)
    r21 = 2 * (y * z + w * x)
    r22 = 1 - 2 * (x * x + y * y)

    es = jnp.exp(scale) * BASE_SCALE
    A = es
    B = shift * es + BASE_SHIFT

    i = pl.program_id(0)
    u = lax.broadcasted_iota(jnp.int32, (BH, W), 1).astype(jnp.float32)
    v = (lax.broadcasted_iota(jnp.int32, (BH, W), 0) + i * BH).astype(jnp.float32)

    d = raw_ref[...] * A + B

    ifx = 1.0 / fx
    ify = 1.0 / fy

    def plane(ra, rb, rc, t):
        c0 = ra * ifx
        c1 = rb * ify
        c2 = rc - ra * ox * ifx - rb * oy * ify
        a = u * c0 + v * c1 + c2
        return d * a + t

    out_ref[0, :, :] = plane(r00, r01, r02, t0)
    out_ref[1, :, :] = plane(r10, r11, r12, t1)
    out_ref[2, :, :] = plane(r20, r21, r22, t2)


def kernel(raw_depth, quaternion, T, scale, shift, Focalx, Focaly, Offsetx, Offsety):
    params = jnp.concatenate([
        quaternion,
        T,
        jnp.stack([scale, shift, Focalx, Focaly, Offsetx, Offsety]),
    ]).astype(jnp.float32)

    planes = pl.pallas_call(
        _body,
        grid=(H // BH,),
        in_specs=[
            pl.BlockSpec(memory_space=pltpu.SMEM),
            pl.BlockSpec((BH, W), lambda i: (i, 0)),
        ],
        out_specs=pl.BlockSpec((3, BH, W), lambda i: (0, i, 0)),
        out_shape=jax.ShapeDtypeStruct((3, H, W), jnp.float32),
    )(params, raw_depth)
    return jnp.transpose(planes, (1, 2, 0))


# speedup vs baseline: 1.0141x; 1.0141x over previous
import jax
import jax.numpy as jnp
from jax import lax
from jax.experimental import pallas as pl
from jax.experimental.pallas import tpu as pltpu

BASE_SCALE = 150.0
BASE_SHIFT = 10.0
H, W = 2160, 3840
BH = 240


def _body(params_ref, raw_ref, out_ref):
    p = params_ref
    q0, q1, q2, q3 = p[0], p[1], p[2], p[3]
    t0, t1, t2 = p[4], p[5], p[6]
    scale, shift = p[7], p[8]
    fx, fy, ox, oy = p[9], p[10], p[11], p[12]

    inv = lax.rsqrt(q0 * q0 + q1 * q1 + q2 * q2 + q3 * q3 + 1e-12)
    w = q0 * inv
    x = q1 * inv
    y = q2 * inv
    z = q3 * inv
    r00 = 1 - 2 * (y * y + z * z)
    r01 = 2 * (x * y - w * z)
    r02 = 2 * (x * z + w * y)
    r10 = 2 * (x * y + w * z)
    r11 = 1 - 2 * (x * x + z * z)
    r12 = 2 * (y * z - w * x)
    r20 = 2 * (x * z - w * y)
    r21 = 2 * (y * z + w * x)
    r22 = 1 - 2 * (x * x + y * y)

    es = jnp.exp(scale) * BASE_SCALE
    A = es
    B = shift * es + BASE_SHIFT

    i = pl.program_id(0)
    u = lax.broadcasted_iota(jnp.int32, (BH, W), 1).astype(jnp.float32)
    v = (lax.broadcasted_iota(jnp.int32, (BH, W), 0) + i * BH).astype(jnp.float32)

    d = raw_ref[...] * A + B

    ifx = 1.0 / fx
    ify = 1.0 / fy

    def plane(ra, rb, rc, t):
        c0 = ra * ifx
        c1 = rb * ify
        c2 = rc - ra * ox * ifx - rb * oy * ify
        a = u * c0 + v * c1 + c2
        return d * a + t

    out_ref[0, :, :] = plane(r00, r01, r02, t0)
    out_ref[1, :, :] = plane(r10, r11, r12, t1)
    out_ref[2, :, :] = plane(r20, r21, r22, t2)


def kernel(raw_depth, quaternion, T, scale, shift, Focalx, Focaly, Offsetx, Offsety):
    params = jnp.concatenate([
        quaternion,
        T,
        jnp.stack([scale, shift, Focalx, Focaly, Offsetx, Offsety]),
    ]).astype(jnp.float32)

    planes = pl.pallas_call(
        _body,
        grid=(H // BH,),
        in_specs=[
            pl.BlockSpec(memory_space=pltpu.SMEM),
            pl.BlockSpec((BH, W), lambda i: (i, 0)),
        ],
        out_specs=pl.BlockSpec((3, BH, W), lambda i: (0, i, 0)),
        out_shape=jax.ShapeDtypeStruct((3, H, W), jnp.float32),
        compiler_params=pltpu.CompilerParams(
            dimension_semantics=("parallel",),
        ),
    )(params, raw_depth)
    return jnp.transpose(planes, (1, 2, 0))


# speedup vs baseline: 1.0457x; 1.0311x over previous
import jax
import jax.numpy as jnp
from jax import lax
from jax.experimental import pallas as pl
from jax.experimental.pallas import tpu as pltpu

BASE_SCALE = 150.0
BASE_SHIFT = 10.0
H, W = 2160, 3840
BH = 240


def _body(params_ref, raw_ref, out_ref):
    p = params_ref
    q0, q1, q2, q3 = p[0], p[1], p[2], p[3]
    t0, t1, t2 = p[4], p[5], p[6]
    scale, shift = p[7], p[8]
    fx, fy, ox, oy = p[9], p[10], p[11], p[12]

    inv = lax.rsqrt(q0 * q0 + q1 * q1 + q2 * q2 + q3 * q3 + 1e-12)
    w = q0 * inv
    x = q1 * inv
    y = q2 * inv
    z = q3 * inv
    r00 = 1 - 2 * (y * y + z * z)
    r01 = 2 * (x * y - w * z)
    r02 = 2 * (x * z + w * y)
    r10 = 2 * (x * y + w * z)
    r11 = 1 - 2 * (x * x + z * z)
    r12 = 2 * (y * z - w * x)
    r20 = 2 * (x * z - w * y)
    r21 = 2 * (y * z + w * x)
    r22 = 1 - 2 * (x * x + y * y)

    es = jnp.exp(scale) * BASE_SCALE
    A = es
    B = shift * es + BASE_SHIFT

    i = pl.program_id(0)
    u1 = lax.broadcasted_iota(jnp.int32, (1, W), 1).astype(jnp.float32)
    v1 = (lax.broadcasted_iota(jnp.int32, (BH, 1), 0) + i * BH).astype(jnp.float32)

    d = raw_ref[...] * A + B

    ifx = 1.0 / fx
    ify = 1.0 / fy

    def plane(ra, rb, rc, t):
        c0 = ra * ifx
        c1 = rb * ify
        c2 = rc - ra * ox * ifx - rb * oy * ify
        a = (u1 * c0 + c2) + v1 * c1
        return d * a + t

    out_ref[0, :, :] = plane(r00, r01, r02, t0)
    out_ref[1, :, :] = plane(r10, r11, r12, t1)
    out_ref[2, :, :] = plane(r20, r21, r22, t2)


def kernel(raw_depth, quaternion, T, scale, shift, Focalx, Focaly, Offsetx, Offsety):
    params = jnp.concatenate([
        quaternion,
        T,
        jnp.stack([scale, shift, Focalx, Focaly, Offsetx, Offsety]),
    ]).astype(jnp.float32)

    planes = pl.pallas_call(
        _body,
        grid=(H // BH,),
        in_specs=[
            pl.BlockSpec(memory_space=pltpu.SMEM),
            pl.BlockSpec((BH, W), lambda i: (i, 0)),
        ],
        out_specs=pl.BlockSpec((3, BH, W), lambda i: (0, i, 0)),
        out_shape=jax.ShapeDtypeStruct((3, H, W), jnp.float32),
        compiler_params=pltpu.CompilerParams(
            dimension_semantics=("parallel",),
        ),
    )(params, raw_depth)
    return jnp.transpose(planes, (1, 2, 0))
